# baseline (speedup 1.0000x reference)
import functools

import jax
import jax.numpy as jnp
import numpy as np
from jax import lax
from jax.experimental import pallas as pl
from jax.experimental.pallas import tpu as pltpu
from jax.experimental.pallas import tpu_sc as plsc

_C = 768
_E = 6
_EP = 16
_T = 2048
_ET = (0, 1, 1, 1, 2, 2)
_BLK = 256
_CHUNK = 64
_NEG = float("-inf")


def _gelu(v):
    return 0.5 * v * (1.0 + jax.lax.erf(v * (2.0 ** -0.5)))


def _silu(v):
    return v * jax.nn.sigmoid(v)


def _lnorm(h, g, b, eps=1e-5):
    m = jnp.mean(h, axis=-1, keepdims=True)
    var = jnp.mean((h - m) * (h - m), axis=-1, keepdims=True)
    return (h - m) / jnp.sqrt(var + eps) * g + b


def _row(v):
    return v.reshape(1, -1)


def _b16(v):
    return v.astype(jnp.bfloat16)


def _prep_body(tf8_ref, w1b_ref, rb1_ref, ce_ref):
    ce_ref[...] = (
        jnp.dot(tf8_ref[...], w1b_ref[...], preferred_element_type=jnp.float32)
        + rb1_ref[...]
    )


def _router_body(x_ref, w1a_ref, ce_ref, rw2_ref, rb2_ref, nw1_ref, nb1_ref,
                 nw2_ref, nb2_ref, norm_ref, bonus_ref, gates_ref, gpad_ref,
                 slot_ref):
    xx = x_ref[...]
    xr = jnp.dot(xx, w1a_ref[...], preferred_element_type=jnp.float32)
    cols = []
    for e in range(_E):
        ge = _gelu(xr + ce_ref[e:e + 1, :])
        lo = jnp.dot(ge, rw2_ref[...],
                     preferred_element_type=jnp.float32) + rb2_ref[...]
        cols.append(jnp.sum(lo, axis=1, keepdims=True) / float(_E))
    cols.append(jnp.zeros((xx.shape[0], _EP - _E), jnp.float32))
    logits = jnp.concatenate(cols, axis=1)

    nh = _gelu(jnp.dot(xx, nw1_ref[...], preferred_element_type=jnp.float32)
               + nb1_ref[...])
    nsc = jax.nn.softplus(jax.nn.softplus(
        jnp.dot(nh, nw2_ref[...], preferred_element_type=jnp.float32)
        + nb2_ref[...]))
    noisy = logits + norm_ref[...] * nsc + bonus_ref[...]

    ii = jax.lax.broadcasted_iota(jnp.int32, noisy.shape, 1)
    m1 = jnp.max(noisy, axis=1, keepdims=True)
    i1 = jnp.min(jnp.where(noisy == m1, ii, _EP), axis=1, keepdims=True)
    mk1 = ii == i1
    n2 = jnp.where(mk1, _NEG, noisy)
    m2 = jnp.max(n2, axis=1, keepdims=True)
    i2 = jnp.min(jnp.where(n2 == m2, ii, _EP), axis=1, keepdims=True)
    mk2 = ii == i2
    s2 = jnp.exp(m2 - m1)
    den = 1.0 + s2
    g = (mk1.astype(jnp.float32) + mk2.astype(jnp.float32) * s2) / den
    gates_ref[...] = g
    gpad_ref[...] = jnp.concatenate(
        [g, jnp.zeros((g.shape[0], 128 - _EP), jnp.float32)], axis=1)
    slot_ref[...] = mk1.astype(jnp.float32)


_TRASH = _T


def _listbuild(gates_flat, slot_flat):
    mesh = plsc.VectorSubcoreMesh(core_axis_name="c", subcore_axis_name="s")

    @functools.partial(
        pl.kernel, mesh=mesh,
        compiler_params=pltpu.CompilerParams(needs_layout_passes=False),
        out_type=[
            jax.ShapeDtypeStruct((_E, _T), jnp.int32),
            jax.ShapeDtypeStruct((_E, _T), jnp.int32),
            jax.ShapeDtypeStruct((_E, _T), jnp.int32),
            jax.ShapeDtypeStruct((_E, 16), jnp.int32),
        ],
        scratch_types=[
            pltpu.VMEM((_T * _EP,), jnp.float32),
            pltpu.VMEM((_T * _EP,), jnp.float32),
            pltpu.VMEM((_T,), jnp.int32),
            pltpu.VMEM((_T,), jnp.int32),
            pltpu.VMEM((_T,), jnp.int32),
            pltpu.VMEM((16,), jnp.int32),
            pltpu.SemaphoreType.DMA,
        ],
    )
    def k(gates_hbm, slot_hbm, ids_hbm, idsf_hbm, idss_hbm, cnt_hbm,
          gates_v, slot_v, ids_v, idsf_v, idss_v, cnt_v, sem):
        wid = lax.axis_index("s") * 2 + lax.axis_index("c")

        @pl.when(wid < _E)
        def _():
            pltpu.async_copy(gates_hbm, gates_v, sem).wait()
            pltpu.async_copy(slot_hbm, slot_v, sem).wait()
            lane = lax.iota(jnp.int32, 16)
            trash = jnp.zeros((16,), jnp.int32) + _TRASH

            def body(i, count):
                ids_v[pl.ds(i * 16, 16)] = jnp.zeros((16,), jnp.int32)
                idsf_v[pl.ds(i * 16, 16)] = trash
                idss_v[pl.ds(i * 16, 16)] = trash
                rows = i * 16 + lane
                g16 = plsc.load_gather(gates_v, [rows * _EP + wid])
                s16 = plsc.load_gather(slot_v, [rows * _EP + wid])
                m = g16 > 0.0
                mi = m.astype(jnp.int32)
                pos = plsc.cumsum(mi) + (count - 1)
                first = s16 > 0.0
                plsc.store_scatter(ids_v, [pos], rows, mask=m)
                plsc.store_scatter(idsf_v, [pos],
                                   jnp.where(first, rows, trash), mask=m)
                plsc.store_scatter(idss_v, [pos],
                                   jnp.where(first, trash, rows), mask=m)
                return count + jnp.sum(mi)

            total = lax.fori_loop(0, _T // 16, body, jnp.int32(0))
            cnt_v[...] = jnp.zeros((16,), jnp.int32) + total
            pltpu.sync_copy(ids_v, ids_hbm.at[wid])
            pltpu.sync_copy(idsf_v, idsf_hbm.at[wid])
            pltpu.sync_copy(idss_v, idss_hbm.at[wid])
            pltpu.sync_copy(cnt_v, cnt_hbm.at[wid])

    return k(gates_flat, slot_flat)


def _gather(x, gates, ids, cnt):
    mesh = plsc.VectorSubcoreMesh(core_axis_name="c", subcore_axis_name="s")
    n_out = [jax.ShapeDtypeStruct((_T, _C), jnp.float32) for _ in range(_E)]
    g_out = [jax.ShapeDtypeStruct((_T, 128), jnp.float32) for _ in range(_E)]

    @functools.partial(
        pl.kernel, mesh=mesh,
        compiler_params=pltpu.CompilerParams(needs_layout_passes=False),
        out_type=n_out + g_out,
        scratch_types=[
            pltpu.VMEM((_CHUNK,), jnp.int32),
            pltpu.VMEM((_CHUNK, _C), jnp.float32),
            pltpu.VMEM((_CHUNK, 128), jnp.float32),
            pltpu.VMEM((16,), jnp.int32),
            pltpu.SemaphoreType.DMA,
        ],
    )
    def k(x_hbm, gates_hbm, ids_hbm, cnt_hbm, *rest):
        outs = rest[:2 * _E]
        idx_v, rows_v, grows_v, cnt_v, sem = rest[2 * _E:]
        wid = lax.axis_index("s") * 2 + lax.axis_index("c")
        for e in range(_E):
            pltpu.sync_copy(cnt_hbm.at[e], cnt_v)
            n = jnp.max(cnt_v[...])

            @pl.when(wid * _CHUNK < n)
            def _():
                pltpu.sync_copy(ids_hbm.at[e, pl.ds(wid * _CHUNK, _CHUNK)],
                                idx_v)
                pltpu.async_copy(x_hbm.at[idx_v], rows_v, sem).wait()
                pltpu.sync_copy(rows_v,
                                outs[e].at[pl.ds(wid * _CHUNK, _CHUNK)])
                pltpu.async_copy(gates_hbm.at[idx_v], grows_v, sem).wait()
                pltpu.sync_copy(grows_v,
                                outs[_E + e].at[pl.ds(wid * _CHUNK, _CHUNK)])

    r = k(x, gates, ids, cnt)
    return r[:_E], r[_E:]


def _combine(idsf, idss, cnt, yes):
    mesh = plsc.VectorSubcoreMesh(core_axis_name="c", subcore_axis_name="s")

    @functools.partial(
        pl.kernel, mesh=mesh,
        compiler_params=pltpu.CompilerParams(needs_layout_passes=False),
        out_type=[
            jax.ShapeDtypeStruct((_T + 8, _C), jnp.float32),
            jax.ShapeDtypeStruct((_T + 8, _C), jnp.float32),
        ],
        scratch_types=[
            pltpu.VMEM((_CHUNK,), jnp.int32),
            pltpu.VMEM((_CHUNK,), jnp.int32),
            pltpu.VMEM((_CHUNK, _C), jnp.float32),
            pltpu.VMEM((16,), jnp.int32),
            pltpu.SemaphoreType.DMA,
        ],
    )
    def k(idsf_hbm, idss_hbm, cnt_hbm, y0, y1, y2, y3, y4, y5, o1_hbm, o2_hbm,
          idxf_v, idxs_v, rows_v, cnt_v, sem):
        ys = (y0, y1, y2, y3, y4, y5)
        wid = lax.axis_index("s") * 2 + lax.axis_index("c")
        for e in range(_E):
            pltpu.sync_copy(cnt_hbm.at[e], cnt_v)
            n = jnp.max(cnt_v[...])

            @pl.when(wid * _CHUNK < n)
            def _():
                pltpu.sync_copy(idsf_hbm.at[e, pl.ds(wid * _CHUNK, _CHUNK)],
                                idxf_v)
                pltpu.sync_copy(idss_hbm.at[e, pl.ds(wid * _CHUNK, _CHUNK)],
                                idxs_v)
                pltpu.sync_copy(ys[e].at[pl.ds(wid * _CHUNK, _CHUNK)], rows_v)
                pltpu.sync_copy(rows_v, o1_hbm.at[idxf_v])
                pltpu.sync_copy(rows_v, o2_hbm.at[idxs_v])

    return k(idsf, idss, cnt, *yes)


def _final_add_body(a_ref, b_ref, o_ref):
    o_ref[...] = a_ref[...] + b_ref[...]


def _mask_rows(y, n, pid, blk):
    ri = jax.lax.broadcasted_iota(jnp.int32, (blk, 1), 0)
    return jnp.where(ri < n - pid * blk, y, 0.0)


def _deep_a_body(cnt_ref, x_ref, w1, b1, w2, b2, lg, lb, out_ref, *, e, blk):
    @pl.when(pl.program_id(0) * blk < cnt_ref[e])
    def _():
        xx = x_ref[0].astype(jnp.bfloat16)
        h = _silu(jnp.dot(xx, w1[...], preferred_element_type=jnp.float32)
                  + b1[...])
        h = jnp.dot(h.astype(jnp.bfloat16), w2[...],
                    preferred_element_type=jnp.float32) + b2[...]
        out_ref[...] = _silu(_lnorm(h, lg[...], lb[...])).astype(jnp.bfloat16)


def _deep_b_body(cnt_ref, x_ref, g_ref, h_ref, w3, b3, ng, nb, out_ref,
                 *, e, blk):
    pid = pl.program_id(0)
    n = cnt_ref[e]

    @pl.when(pid * blk < n)
    def _():
        xx = x_ref[0]
        o = jnp.dot(h_ref[...], w3[...],
                    preferred_element_type=jnp.float32) + b3[...]
        y = _lnorm(xx + o, ng[...], nb[...])
        y = g_ref[0][:, e:e + 1] * y
        out_ref[...] = _mask_rows(y, n, pid, blk)


def _wide_body(cnt_ref, x_ref, g_ref, w1, b1, lg, lb, w2, b2, ng, nb, out_ref,
               *, e, blk):
    pid = pl.program_id(0)
    n = cnt_ref[e]

    @pl.when(pid * blk < n)
    def _():
        xx = x_ref[0]
        h = _gelu(jnp.dot(xx.astype(jnp.bfloat16), w1[...],
                          preferred_element_type=jnp.float32) + b1[...])
        h = _lnorm(h, lg[...], lb[...])
        o = jnp.dot(h.astype(jnp.bfloat16), w2[...],
                    preferred_element_type=jnp.float32) + b2[...]
        y = _lnorm(xx + o, ng[...], nb[...])
        y = g_ref[0][:, e:e + 1] * y
        out_ref[...] = _mask_rows(y, n, pid, blk)


def _hybrid_body(cnt_ref, x_ref, g_ref, p1w1, p1b1, p1w2, p1b2,
                 p2w1, p2b1, p2w2, p2b2, pw1, pw2, pb, ng, nb, out_ref,
                 *, e, blk):
    pid = pl.program_id(0)
    n = cnt_ref[e]

    @pl.when(pid * blk < n)
    def _():
        xx = x_ref[0]
        x16 = xx.astype(jnp.bfloat16)
        h1 = _gelu(jnp.dot(x16, p1w1[...], preferred_element_type=jnp.float32)
                   + p1b1[...])
        o1 = jnp.dot(h1.astype(jnp.bfloat16), p1w2[...],
                     preferred_element_type=jnp.float32) + p1b2[...]
        h2 = _silu(jnp.dot(x16, p2w1[...], preferred_element_type=jnp.float32)
                   + p2b1[...])
        o2 = jnp.dot(h2.astype(jnp.bfloat16), p2w2[...],
                     preferred_element_type=jnp.float32) + p2b2[...]
        o = (jnp.dot(o1.astype(jnp.bfloat16), pw1[...],
                     preferred_element_type=jnp.float32)
             + jnp.dot(o2.astype(jnp.bfloat16), pw2[...],
                       preferred_element_type=jnp.float32) + pb[...])
        y = _lnorm(xx + o, ng[...], nb[...])
        y = g_ref[0][:, e:e + 1] * y
        out_ref[...] = _mask_rows(y, n, pid, blk)


def _ragged_call(body, e, cntv, xg, gg, weights, out_shape, blk,
                 extra=None, with_g=True, out_dtype=jnp.float32):
    ins = [xg.reshape(1, *xg.shape)]
    specs = [pl.BlockSpec((1, blk, _C), lambda t, s: (0, t, 0))]
    if with_g:
        ins.append(gg.reshape(1, *gg.shape))
        specs.append(pl.BlockSpec((1, blk, 128), lambda t, s: (0, t, 0)))
    if extra is not None:
        ins.append(extra[0])
        specs.append(pl.BlockSpec((blk, extra[1]), lambda t, s: (t, 0)))
    for w in weights:
        ins.append(w)
        specs.append(pl.BlockSpec(w.shape, lambda t, s, n=w.ndim: (0,) * n))
    grid_spec = pltpu.PrefetchScalarGridSpec(
        num_scalar_prefetch=1,
        grid=(_T // blk,),
        in_specs=specs,
        out_specs=pl.BlockSpec((blk, out_shape[1]), lambda t, s: (t, 0)),
    )
    return pl.pallas_call(
        functools.partial(body, e=e, blk=blk),
        grid_spec=grid_spec,
        out_shape=jax.ShapeDtypeStruct(out_shape, out_dtype),
    )(cntv, *ins)


def kernel(x, params):
    p = params
    xf = x.reshape(_T, _C)
    et = np.array(_ET)

    tf = p["type_emb2"][jnp.array(et, jnp.int32)]
    tf8 = jnp.concatenate([tf, jnp.zeros((8 - _E, 2 * _C), jnp.float32)], 0)
    w1a = p["route_w1"][:_C]
    w1b = p["route_w1"][_C:]
    rb1 = _row(p["route_b1"])
    rw2p = jnp.zeros((4 * _C, _EP), jnp.float32).at[:, :_E].set(p["route_w2"])
    rb2p = jnp.zeros((1, _EP), jnp.float32).at[0, :_E].set(p["route_b2"])
    temp = jnp.clip(p["temperature"] * (0.95 ** (_T // 100)), 0.05, 3.0)
    norm = jax.random.normal(jax.random.key(42), (_T, _E), jnp.float32)
    norm_p = jnp.concatenate(
        [temp * norm, jnp.zeros((_T, _EP - _E), jnp.float32)], 1)
    bonus = jnp.full((_EP,), _NEG, jnp.float32)
    bonus = bonus.at[:_E].set(0.3 * (et == 1).astype(jnp.float32))
    bonus = _row(bonus)
    nw1p = jnp.zeros((_C, 128), jnp.float32).at[:, :2 * _E].set(p["noise_w1"])
    nb1p = jnp.zeros((1, 128), jnp.float32).at[0, :2 * _E].set(p["noise_b1"])
    nw2p = jnp.zeros((128, _EP), jnp.float32).at[:2 * _E, :_E].set(p["noise_w2"])
    nb2p = jnp.zeros((1, _EP), jnp.float32).at[0, :_E].set(p["noise_b2"])

    ce = pl.pallas_call(
        _prep_body,
        out_shape=jax.ShapeDtypeStruct((8, 4 * _C), jnp.float32),
    )(tf8, w1b, rb1)

    gates, gpad, slot1 = pl.pallas_call(
        _router_body,
        grid=(_T // _BLK,),
        in_specs=[
            pl.BlockSpec((_BLK, _C), lambda t: (t, 0)),
            pl.BlockSpec((_C, 4 * _C), lambda t: (0, 0)),
            pl.BlockSpec((8, 4 * _C), lambda t: (0, 0)),
            pl.BlockSpec((4 * _C, _EP), lambda t: (0, 0)),
            pl.BlockSpec((1, _EP), lambda t: (0, 0)),
            pl.BlockSpec((_C, 128), lambda t: (0, 0)),
            pl.BlockSpec((1, 128), lambda t: (0, 0)),
            pl.BlockSpec((128, _EP), lambda t: (0, 0)),
            pl.BlockSpec((1, _EP), lambda t: (0, 0)),
            pl.BlockSpec((_BLK, _EP), lambda t: (t, 0)),
            pl.BlockSpec((1, _EP), lambda t: (0, 0)),
        ],
        out_specs=[
            pl.BlockSpec((_BLK, _EP), lambda t: (t, 0)),
            pl.BlockSpec((_BLK, 128), lambda t: (t, 0)),
            pl.BlockSpec((_BLK, _EP), lambda t: (t, 0)),
        ],
        out_shape=[
            jax.ShapeDtypeStruct((_T, _EP), jnp.float32),
            jax.ShapeDtypeStruct((_T, 128), jnp.float32),
            jax.ShapeDtypeStruct((_T, _EP), jnp.float32),
        ],
    )(xf, w1a, ce, rw2p, rb2p, nw1p, nb1p, nw2p, nb2p, norm_p, bonus)

    ids, idsf, idss, cnt = _listbuild(gates.reshape(-1), slot1.reshape(-1))
    xgs, ggs = _gather(xf, gpad, ids, cnt)
    cntv = cnt[:, 0]

    ex = p["experts"]
    yes = []
    for e, t in enumerate(_ET):
        q = ex[e]
        if t == 0:
            hmid = _ragged_call(
                _deep_a_body, e, cntv, xgs[e], None,
                [_b16(q["w1"]), _row(q["b1"]), _b16(q["w2"]), _row(q["b2"]),
                 _row(q["ln_g"]), _row(q["ln_b"])],
                (_T, 4 * _C), 128, with_g=False, out_dtype=jnp.bfloat16)
            ye = _ragged_call(
                _deep_b_body, e, cntv, xgs[e], ggs[e],
                [_b16(q["w3"]), _row(q["b3"]), _row(q["ng"]), _row(q["nb"])],
                (_T, _C), _BLK, extra=(hmid, 4 * _C))
        elif t == 1:
            ye = _ragged_call(
                _wide_body, e, cntv, xgs[e], ggs[e],
                [_b16(q["w1"]), _row(q["b1"]),
                 _row(q["ln_g"]), _row(q["ln_b"]),
                 _b16(q["w2"]), _row(q["b2"]),
                 _row(q["ng"]), _row(q["nb"])],
                (_T, _C), _BLK)
        else:
            ye = _ragged_call(
                _hybrid_body, e, cntv, xgs[e], ggs[e],
                [_b16(q["p1w1"]), _row(q["p1b1"]),
                 _b16(q["p1w2"]), _row(q["p1b2"]),
                 _b16(q["p2w1"]), _row(q["p2b1"]),
                 _b16(q["p2w2"]), _row(q["p2b2"]),
                 _b16(q["proj_w"][:_C]), _b16(q["proj_w"][_C:]),
                 _row(q["proj_b"]),
                 _row(q["ng"]), _row(q["nb"])],
                (_T, _C), _BLK)
        yes.append(ye)

    o1, o2 = _combine(idsf, idss, cnt, yes)
    out = pl.pallas_call(
        _final_add_body,
        grid=(_T // 512,),
        in_specs=[
            pl.BlockSpec((512, _C), lambda t: (t, 0)),
            pl.BlockSpec((512, _C), lambda t: (t, 0)),
        ],
        out_specs=pl.BlockSpec((512, _C), lambda t: (t, 0)),
        out_shape=jax.ShapeDtypeStruct((_T, _C), jnp.float32),
    )(o1, o2)

    return out.reshape(1, _T, _C)


# speedup vs baseline: 1.0011x; 1.0011x over previous
import functools

import jax
import jax.numpy as jnp
import numpy as np
from jax import lax
from jax.experimental import pallas as pl
from jax.experimental.pallas import tpu as pltpu
from jax.experimental.pallas import tpu_sc as plsc

_C = 768
_E = 6
_EP = 16
_T = 2048
_ET = (0, 1, 1, 1, 2, 2)
_BLK = 256
_CHUNK = 64
_NEG = float("-inf")


def _gelu(v):
    return 0.5 * v * (1.0 + jax.lax.erf(v * (2.0 ** -0.5)))


def _silu(v):
    return v * jax.nn.sigmoid(v)


def _lnorm(h, g, b, eps=1e-5):
    m = jnp.mean(h, axis=-1, keepdims=True)
    var = jnp.mean((h - m) * (h - m), axis=-1, keepdims=True)
    return (h - m) / jnp.sqrt(var + eps) * g + b


def _row(v):
    return v.reshape(1, -1)


def _b16(v):
    return v.astype(jnp.bfloat16)


def _prep_body(tf8_ref, w1b_ref, rb1_ref, ce_ref):
    ce_ref[...] = (
        jnp.dot(tf8_ref[...], w1b_ref[...], preferred_element_type=jnp.float32)
        + rb1_ref[...]
    )


def _router_body(x_ref, w1a_ref, ce_ref, rw2_ref, rb2_ref, nw1_ref, nb1_ref,
                 nw2_ref, nb2_ref, norm_ref, bonus_ref, gates_ref, gpad_ref,
                 slot_ref):
    xx = x_ref[...]
    xr = jnp.dot(xx, w1a_ref[...], preferred_element_type=jnp.float32)
    cols = []
    for e in range(_E):
        ge = _gelu(xr + ce_ref[e:e + 1, :])
        lo = jnp.dot(ge, rw2_ref[...],
                     preferred_element_type=jnp.float32) + rb2_ref[...]
        cols.append(jnp.sum(lo, axis=1, keepdims=True) / float(_E))
    cols.append(jnp.zeros((xx.shape[0], _EP - _E), jnp.float32))
    logits = jnp.concatenate(cols, axis=1)

    nh = _gelu(jnp.dot(xx, nw1_ref[...], preferred_element_type=jnp.float32)
               + nb1_ref[...])
    nsc = jax.nn.softplus(jax.nn.softplus(
        jnp.dot(nh, nw2_ref[...], preferred_element_type=jnp.float32)
        + nb2_ref[...]))
    noisy = logits + norm_ref[...] * nsc + bonus_ref[...]

    ii = jax.lax.broadcasted_iota(jnp.int32, noisy.shape, 1)
    m1 = jnp.max(noisy, axis=1, keepdims=True)
    i1 = jnp.min(jnp.where(noisy == m1, ii, _EP), axis=1, keepdims=True)
    mk1 = ii == i1
    n2 = jnp.where(mk1, _NEG, noisy)
    m2 = jnp.max(n2, axis=1, keepdims=True)
    i2 = jnp.min(jnp.where(n2 == m2, ii, _EP), axis=1, keepdims=True)
    mk2 = ii == i2
    s2 = jnp.exp(m2 - m1)
    den = 1.0 + s2
    g = (mk1.astype(jnp.float32) + mk2.astype(jnp.float32) * s2) / den
    gates_ref[...] = g
    gpad_ref[...] = jnp.concatenate(
        [g, jnp.zeros((g.shape[0], 128 - _EP), jnp.float32)], axis=1)
    slot_ref[...] = mk1.astype(jnp.float32)


_TRASH = _T


def _listbuild(gates_flat, slot_flat):
    mesh = plsc.VectorSubcoreMesh(core_axis_name="c", subcore_axis_name="s")

    @functools.partial(
        pl.kernel, mesh=mesh,
        compiler_params=pltpu.CompilerParams(needs_layout_passes=False),
        out_type=[
            jax.ShapeDtypeStruct((_E, _T), jnp.int32),
            jax.ShapeDtypeStruct((_E, _T), jnp.int32),
            jax.ShapeDtypeStruct((_E, _T), jnp.int32),
            jax.ShapeDtypeStruct((_E, 16), jnp.int32),
        ],
        scratch_types=[
            pltpu.VMEM((_T * _EP,), jnp.float32),
            pltpu.VMEM((_T * _EP,), jnp.float32),
            pltpu.VMEM((_T,), jnp.int32),
            pltpu.VMEM((_T,), jnp.int32),
            pltpu.VMEM((_T,), jnp.int32),
            pltpu.VMEM((16,), jnp.int32),
            pltpu.SemaphoreType.DMA,
        ],
    )
    def k(gates_hbm, slot_hbm, ids_hbm, idsf_hbm, idss_hbm, cnt_hbm,
          gates_v, slot_v, ids_v, idsf_v, idss_v, cnt_v, sem):
        wid = lax.axis_index("s") * 2 + lax.axis_index("c")

        @pl.when(wid < _E)
        def _():
            pltpu.async_copy(gates_hbm, gates_v, sem).wait()
            pltpu.async_copy(slot_hbm, slot_v, sem).wait()
            lane = lax.iota(jnp.int32, 16)
            trash = jnp.zeros((16,), jnp.int32) + _TRASH

            def body(i, count):
                ids_v[pl.ds(i * 16, 16)] = jnp.zeros((16,), jnp.int32)
                idsf_v[pl.ds(i * 16, 16)] = trash
                idss_v[pl.ds(i * 16, 16)] = trash
                rows = i * 16 + lane
                g16 = plsc.load_gather(gates_v, [rows * _EP + wid])
                s16 = plsc.load_gather(slot_v, [rows * _EP + wid])
                m = g16 > 0.0
                mi = m.astype(jnp.int32)
                pos = plsc.cumsum(mi) + (count - 1)
                first = s16 > 0.0
                plsc.store_scatter(ids_v, [pos], rows, mask=m)
                plsc.store_scatter(idsf_v, [pos],
                                   jnp.where(first, rows, trash), mask=m)
                plsc.store_scatter(idss_v, [pos],
                                   jnp.where(first, trash, rows), mask=m)
                return count + jnp.sum(mi)

            total = lax.fori_loop(0, _T // 16, body, jnp.int32(0))
            cnt_v[...] = jnp.zeros((16,), jnp.int32) + total
            pltpu.sync_copy(ids_v, ids_hbm.at[wid])
            pltpu.sync_copy(idsf_v, idsf_hbm.at[wid])
            pltpu.sync_copy(idss_v, idss_hbm.at[wid])
            pltpu.sync_copy(cnt_v, cnt_hbm.at[wid])

    return k(gates_flat, slot_flat)


def _gather(x, gates, ids, cnt):
    mesh = plsc.VectorSubcoreMesh(core_axis_name="c", subcore_axis_name="s")
    n_out = [jax.ShapeDtypeStruct((_T, _C), jnp.float32) for _ in range(_E)]
    g_out = [jax.ShapeDtypeStruct((_T, 128), jnp.float32) for _ in range(_E)]

    @functools.partial(
        pl.kernel, mesh=mesh,
        compiler_params=pltpu.CompilerParams(needs_layout_passes=False),
        out_type=n_out + g_out,
        scratch_types=[
            pltpu.VMEM((_CHUNK,), jnp.int32),
            pltpu.VMEM((_CHUNK, _C), jnp.float32),
            pltpu.VMEM((_CHUNK, 128), jnp.float32),
            pltpu.VMEM((16,), jnp.int32),
            pltpu.SemaphoreType.DMA,
        ],
    )
    def k(x_hbm, gates_hbm, ids_hbm, cnt_hbm, *rest):
        outs = rest[:2 * _E]
        idx_v, rows_v, grows_v, cnt_v, sem = rest[2 * _E:]
        wid = lax.axis_index("s") * 2 + lax.axis_index("c")
        for e in range(_E):
            pltpu.sync_copy(cnt_hbm.at[e], cnt_v)
            n = jnp.max(cnt_v[...])

            @pl.when(wid * _CHUNK < n)
            def _():
                pltpu.sync_copy(ids_hbm.at[e, pl.ds(wid * _CHUNK, _CHUNK)],
                                idx_v)
                pltpu.async_copy(x_hbm.at[idx_v], rows_v, sem).wait()
                pltpu.sync_copy(rows_v,
                                outs[e].at[pl.ds(wid * _CHUNK, _CHUNK)])
                pltpu.async_copy(gates_hbm.at[idx_v], grows_v, sem).wait()
                pltpu.sync_copy(grows_v,
                                outs[_E + e].at[pl.ds(wid * _CHUNK, _CHUNK)])

    r = k(x, gates, ids, cnt)
    return r[:_E], r[_E:]


def _combine(idsf, idss, cnt, yes):
    mesh = plsc.VectorSubcoreMesh(core_axis_name="c", subcore_axis_name="s")

    @functools.partial(
        pl.kernel, mesh=mesh,
        compiler_params=pltpu.CompilerParams(needs_layout_passes=False),
        out_type=[
            jax.ShapeDtypeStruct((_T + 8, _C), jnp.float32),
            jax.ShapeDtypeStruct((_T + 8, _C), jnp.float32),
        ],
        scratch_types=[
            pltpu.VMEM((_CHUNK,), jnp.int32),
            pltpu.VMEM((_CHUNK,), jnp.int32),
            pltpu.VMEM((_CHUNK, _C), jnp.float32),
            pltpu.VMEM((16,), jnp.int32),
            pltpu.SemaphoreType.DMA,
            pltpu.SemaphoreType.DMA,
            pltpu.SemaphoreType.DMA,
        ],
    )
    def k(idsf_hbm, idss_hbm, cnt_hbm, y0, y1, y2, y3, y4, y5, o1_hbm, o2_hbm,
          idxf_v, idxs_v, rows_v, cnt_v, sem, sem2, sem3):
        ys = (y0, y1, y2, y3, y4, y5)
        wid = lax.axis_index("s") * 2 + lax.axis_index("c")
        for e in range(_E):
            pltpu.sync_copy(cnt_hbm.at[e], cnt_v)
            n = jnp.max(cnt_v[...])

            @pl.when(wid * _CHUNK < n)
            def _():
                cf = pltpu.async_copy(
                    idsf_hbm.at[e, pl.ds(wid * _CHUNK, _CHUNK)], idxf_v, sem)
                cs = pltpu.async_copy(
                    idss_hbm.at[e, pl.ds(wid * _CHUNK, _CHUNK)], idxs_v, sem2)
                cy = pltpu.async_copy(
                    ys[e].at[pl.ds(wid * _CHUNK, _CHUNK)], rows_v, sem3)
                cf.wait()
                cs.wait()
                cy.wait()
                w1 = pltpu.async_copy(rows_v, o1_hbm.at[idxf_v], sem)
                w2 = pltpu.async_copy(rows_v, o2_hbm.at[idxs_v], sem2)
                w1.wait()
                w2.wait()

    return k(idsf, idss, cnt, *yes)


def _final_add_body(a_ref, b_ref, o_ref):
    o_ref[...] = a_ref[...] + b_ref[...]


def _mask_rows(y, n, pid, blk):
    ri = jax.lax.broadcasted_iota(jnp.int32, (blk, 1), 0)
    return jnp.where(ri < n - pid * blk, y, 0.0)


def _deep_a_body(cnt_ref, x_ref, w1, b1, w2, b2, lg, lb, out_ref, *, e, blk):
    @pl.when(pl.program_id(0) * blk < cnt_ref[e])
    def _():
        xx = x_ref[0].astype(jnp.bfloat16)
        h = _silu(jnp.dot(xx, w1[...], preferred_element_type=jnp.float32)
                  + b1[...])
        h = jnp.dot(h.astype(jnp.bfloat16), w2[...],
                    preferred_element_type=jnp.float32) + b2[...]
        out_ref[...] = _silu(_lnorm(h, lg[...], lb[...])).astype(jnp.bfloat16)


def _deep_b_body(cnt_ref, x_ref, g_ref, h_ref, w3, b3, ng, nb, out_ref,
                 *, e, blk):
    pid = pl.program_id(0)
    n = cnt_ref[e]

    @pl.when(pid * blk < n)
    def _():
        xx = x_ref[0]
        o = jnp.dot(h_ref[...], w3[...],
                    preferred_element_type=jnp.float32) + b3[...]
        y = _lnorm(xx + o, ng[...], nb[...])
        y = g_ref[0][:, e:e + 1] * y
        out_ref[...] = _mask_rows(y, n, pid, blk)


def _wide_body(cnt_ref, x_ref, g_ref, w1, b1, lg, lb, w2, b2, ng, nb, out_ref,
               *, e, blk):
    pid = pl.program_id(0)
    n = cnt_ref[e]

    @pl.when(pid * blk < n)
    def _():
        xx = x_ref[0]
        h = _gelu(jnp.dot(xx.astype(jnp.bfloat16), w1[...],
                          preferred_element_type=jnp.float32) + b1[...])
        h = _lnorm(h, lg[...], lb[...])
        o = jnp.dot(h.astype(jnp.bfloat16), w2[...],
                    preferred_element_type=jnp.float32) + b2[...]
        y = _lnorm(xx + o, ng[...], nb[...])
        y = g_ref[0][:, e:e + 1] * y
        out_ref[...] = _mask_rows(y, n, pid, blk)


def _hybrid_body(cnt_ref, x_ref, g_ref, p1w1, p1b1, p1w2, p1b2,
                 p2w1, p2b1, p2w2, p2b2, pw1, pw2, pb, ng, nb, out_ref,
                 *, e, blk):
    pid = pl.program_id(0)
    n = cnt_ref[e]

    @pl.when(pid * blk < n)
    def _():
        xx = x_ref[0]
        x16 = xx.astype(jnp.bfloat16)
        h1 = _gelu(jnp.dot(x16, p1w1[...], preferred_element_type=jnp.float32)
                   + p1b1[...])
        o1 = jnp.dot(h1.astype(jnp.bfloat16), p1w2[...],
                     preferred_element_type=jnp.float32) + p1b2[...]
        h2 = _silu(jnp.dot(x16, p2w1[...], preferred_element_type=jnp.float32)
                   + p2b1[...])
        o2 = jnp.dot(h2.astype(jnp.bfloat16), p2w2[...],
                     preferred_element_type=jnp.float32) + p2b2[...]
        o = (jnp.dot(o1.astype(jnp.bfloat16), pw1[...],
                     preferred_element_type=jnp.float32)
             + jnp.dot(o2.astype(jnp.bfloat16), pw2[...],
                       preferred_element_type=jnp.float32) + pb[...])
        y = _lnorm(xx + o, ng[...], nb[...])
        y = g_ref[0][:, e:e + 1] * y
        out_ref[...] = _mask_rows(y, n, pid, blk)


def _ragged_call(body, e, cntv, xg, gg, weights, out_shape, blk,
                 extra=None, with_g=True, out_dtype=jnp.float32):
    ins = [xg.reshape(1, *xg.shape)]
    specs = [pl.BlockSpec((1, blk, _C), lambda t, s: (0, t, 0))]
    if with_g:
        ins.append(gg.reshape(1, *gg.shape))
        specs.append(pl.BlockSpec((1, blk, 128), lambda t, s: (0, t, 0)))
    if extra is not None:
        ins.append(extra[0])
        specs.append(pl.BlockSpec((blk, extra[1]), lambda t, s: (t, 0)))
    for w in weights:
        ins.append(w)
        specs.append(pl.BlockSpec(w.shape, lambda t, s, n=w.ndim: (0,) * n))
    grid_spec = pltpu.PrefetchScalarGridSpec(
        num_scalar_prefetch=1,
        grid=(_T // blk,),
        in_specs=specs,
        out_specs=pl.BlockSpec((blk, out_shape[1]), lambda t, s: (t, 0)),
    )
    return pl.pallas_call(
        functools.partial(body, e=e, blk=blk),
        grid_spec=grid_spec,
        out_shape=jax.ShapeDtypeStruct(out_shape, out_dtype),
    )(cntv, *ins)


def kernel(x, params):
    p = params
    xf = x.reshape(_T, _C)
    et = np.array(_ET)

    tf = p["type_emb2"][jnp.array(et, jnp.int32)]
    tf8 = jnp.concatenate([tf, jnp.zeros((8 - _E, 2 * _C), jnp.float32)], 0)
    w1a = p["route_w1"][:_C]
    w1b = p["route_w1"][_C:]
    rb1 = _row(p["route_b1"])
    rw2p = jnp.zeros((4 * _C, _EP), jnp.float32).at[:, :_E].set(p["route_w2"])
    rb2p = jnp.zeros((1, _EP), jnp.float32).at[0, :_E].set(p["route_b2"])
    temp = jnp.clip(p["temperature"] * (0.95 ** (_T // 100)), 0.05, 3.0)
    norm = jax.random.normal(jax.random.key(42), (_T, _E), jnp.float32)
    norm_p = jnp.concatenate(
        [temp * norm, jnp.zeros((_T, _EP - _E), jnp.float32)], 1)
    bonus = jnp.full((_EP,), _NEG, jnp.float32)
    bonus = bonus.at[:_E].set(0.3 * (et == 1).astype(jnp.float32))
    bonus = _row(bonus)
    nw1p = jnp.zeros((_C, 128), jnp.float32).at[:, :2 * _E].set(p["noise_w1"])
    nb1p = jnp.zeros((1, 128), jnp.float32).at[0, :2 * _E].set(p["noise_b1"])
    nw2p = jnp.zeros((128, _EP), jnp.float32).at[:2 * _E, :_E].set(p["noise_w2"])
    nb2p = jnp.zeros((1, _EP), jnp.float32).at[0, :_E].set(p["noise_b2"])

    ce = pl.pallas_call(
        _prep_body,
        out_shape=jax.ShapeDtypeStruct((8, 4 * _C), jnp.float32),
    )(tf8, w1b, rb1)

    gates, gpad, slot1 = pl.pallas_call(
        _router_body,
        grid=(_T // _BLK,),
        in_specs=[
            pl.BlockSpec((_BLK, _C), lambda t: (t, 0)),
            pl.BlockSpec((_C, 4 * _C), lambda t: (0, 0)),
            pl.BlockSpec((8, 4 * _C), lambda t: (0, 0)),
            pl.BlockSpec((4 * _C, _EP), lambda t: (0, 0)),
            pl.BlockSpec((1, _EP), lambda t: (0, 0)),
            pl.BlockSpec((_C, 128), lambda t: (0, 0)),
            pl.BlockSpec((1, 128), lambda t: (0, 0)),
            pl.BlockSpec((128, _EP), lambda t: (0, 0)),
            pl.BlockSpec((1, _EP), lambda t: (0, 0)),
            pl.BlockSpec((_BLK, _EP), lambda t: (t, 0)),
            pl.BlockSpec((1, _EP), lambda t: (0, 0)),
        ],
        out_specs=[
            pl.BlockSpec((_BLK, _EP), lambda t: (t, 0)),
            pl.BlockSpec((_BLK, 128), lambda t: (t, 0)),
            pl.BlockSpec((_BLK, _EP), lambda t: (t, 0)),
        ],
        out_shape=[
            jax.ShapeDtypeStruct((_T, _EP), jnp.float32),
            jax.ShapeDtypeStruct((_T, 128), jnp.float32),
            jax.ShapeDtypeStruct((_T, _EP), jnp.float32),
        ],
    )(xf, w1a, ce, rw2p, rb2p, nw1p, nb1p, nw2p, nb2p, norm_p, bonus)

    ids, idsf, idss, cnt = _listbuild(gates.reshape(-1), slot1.reshape(-1))
    xgs, ggs = _gather(xf, gpad, ids, cnt)
    cntv = cnt[:, 0]

    ex = p["experts"]
    yes = []
    for e, t in enumerate(_ET):
        q = ex[e]
        if t == 0:
            hmid = _ragged_call(
                _deep_a_body, e, cntv, xgs[e], None,
                [_b16(q["w1"]), _row(q["b1"]), _b16(q["w2"]), _row(q["b2"]),
                 _row(q["ln_g"]), _row(q["ln_b"])],
                (_T, 4 * _C), 128, with_g=False, out_dtype=jnp.bfloat16)
            ye = _ragged_call(
                _deep_b_body, e, cntv, xgs[e], ggs[e],
                [_b16(q["w3"]), _row(q["b3"]), _row(q["ng"]), _row(q["nb"])],
                (_T, _C), _BLK, extra=(hmid, 4 * _C))
        elif t == 1:
            ye = _ragged_call(
                _wide_body, e, cntv, xgs[e], ggs[e],
                [_b16(q["w1"]), _row(q["b1"]),
                 _row(q["ln_g"]), _row(q["ln_b"]),
                 _b16(q["w2"]), _row(q["b2"]),
                 _row(q["ng"]), _row(q["nb"])],
                (_T, _C), _BLK)
        else:
            ye = _ragged_call(
                _hybrid_body, e, cntv, xgs[e], ggs[e],
                [_b16(q["p1w1"]), _row(q["p1b1"]),
                 _b16(q["p1w2"]), _row(q["p1b2"]),
                 _b16(q["p2w1"]), _row(q["p2b1"]),
                 _b16(q["p2w2"]), _row(q["p2b2"]),
                 _b16(q["proj_w"][:_C]), _b16(q["proj_w"][_C:]),
                 _row(q["proj_b"]),
                 _row(q["ng"]), _row(q["nb"])],
                (_T, _C), _BLK)
        yes.append(ye)

    o1, o2 = _combine(idsf, idss, cnt, yes)
    out = pl.pallas_call(
        _final_add_body,
        grid=(_T // 512,),
        in_specs=[
            pl.BlockSpec((512, _C), lambda t: (t, 0)),
            pl.BlockSpec((512, _C), lambda t: (t, 0)),
        ],
        out_specs=pl.BlockSpec((512, _C), lambda t: (t, 0)),
        out_shape=jax.ShapeDtypeStruct((_T, _C), jnp.float32),
    )(o1, o2)

    return out.reshape(1, _T, _C)


# speedup vs baseline: 1.3382x; 1.3368x over previous
import functools

import jax
import jax.numpy as jnp
import numpy as np
from jax import lax
from jax.experimental import pallas as pl
from jax.experimental.pallas import tpu as pltpu
from jax.experimental.pallas import tpu_sc as plsc

_C = 768
_E = 6
_EP = 16
_T = 2048
_ET = (0, 1, 1, 1, 2, 2)
_BLK = 256
_CHUNK = 64
_NEG = float("-inf")


def _gelu(v):
    return 0.5 * v * (1.0 + jax.lax.erf(v * (2.0 ** -0.5)))


def _silu(v):
    return v * jax.nn.sigmoid(v)


def _lnorm(h, g, b, eps=1e-5):
    m = jnp.mean(h, axis=-1, keepdims=True)
    var = jnp.mean((h - m) * (h - m), axis=-1, keepdims=True)
    return (h - m) / jnp.sqrt(var + eps) * g + b


def _row(v):
    return v.reshape(1, -1)


def _b16(v):
    return v.astype(jnp.bfloat16)


def _prep_body(tf8_ref, w1b_ref, rb1_ref, ce_ref):
    ce_ref[...] = (
        jnp.dot(tf8_ref[...], w1b_ref[...], preferred_element_type=jnp.float32)
        + rb1_ref[...]
    )


def _router_body(x_ref, w1a_ref, ce_ref, rw2_ref, rb2_ref, nw1_ref, nb1_ref,
                 nw2_ref, nb2_ref, norm_ref, bonus_ref, gates_ref, gpad_ref,
                 slot_ref):
    xx = x_ref[...]
    xr = jnp.dot(xx, w1a_ref[...], preferred_element_type=jnp.float32)
    cols = []
    for e in range(_E):
        ge = _gelu(xr + ce_ref[e:e + 1, :])
        lo = jnp.dot(ge, rw2_ref[...],
                     preferred_element_type=jnp.float32) + rb2_ref[...]
        cols.append(jnp.sum(lo, axis=1, keepdims=True) / float(_E))
    cols.append(jnp.zeros((xx.shape[0], _EP - _E), jnp.float32))
    logits = jnp.concatenate(cols, axis=1)

    nh = _gelu(jnp.dot(xx, nw1_ref[...], preferred_element_type=jnp.float32)
               + nb1_ref[...])
    nsc = jax.nn.softplus(jax.nn.softplus(
        jnp.dot(nh, nw2_ref[...], preferred_element_type=jnp.float32)
        + nb2_ref[...]))
    noisy = logits + norm_ref[...] * nsc + bonus_ref[...]

    ii = jax.lax.broadcasted_iota(jnp.int32, noisy.shape, 1)
    m1 = jnp.max(noisy, axis=1, keepdims=True)
    i1 = jnp.min(jnp.where(noisy == m1, ii, _EP), axis=1, keepdims=True)
    mk1 = ii == i1
    n2 = jnp.where(mk1, _NEG, noisy)
    m2 = jnp.max(n2, axis=1, keepdims=True)
    i2 = jnp.min(jnp.where(n2 == m2, ii, _EP), axis=1, keepdims=True)
    mk2 = ii == i2
    s2 = jnp.exp(m2 - m1)
    den = 1.0 + s2
    g = (mk1.astype(jnp.float32) + mk2.astype(jnp.float32) * s2) / den
    gates_ref[...] = g
    gpad_ref[...] = jnp.concatenate(
        [g, jnp.zeros((g.shape[0], 128 - _EP), jnp.float32)], axis=1)
    slot_ref[...] = mk1.astype(jnp.float32)


_SENT = _E * _T


def _listbuild(gates_flat, slot_flat):
    mesh = plsc.VectorSubcoreMesh(core_axis_name="c", subcore_axis_name="s")

    @functools.partial(
        pl.kernel, mesh=mesh,
        compiler_params=pltpu.CompilerParams(needs_layout_passes=False),
        out_type=[
            jax.ShapeDtypeStruct((_E, _T), jnp.int32),
            jax.ShapeDtypeStruct((_E, _T), jnp.int32),
            jax.ShapeDtypeStruct((_E, _T), jnp.int32),
            jax.ShapeDtypeStruct((_E, 16), jnp.int32),
        ],
        scratch_types=[
            pltpu.VMEM((_T * _EP,), jnp.float32),
            pltpu.VMEM((_T * _EP,), jnp.float32),
            pltpu.VMEM((_T,), jnp.int32),
            pltpu.VMEM((_T,), jnp.int32),
            pltpu.VMEM((_T,), jnp.int32),
            pltpu.VMEM((16,), jnp.int32),
            pltpu.SemaphoreType.DMA,
        ],
    )
    def k(gates_hbm, slot_hbm, ids_hbm, posa_hbm, posb_hbm, cnt_hbm,
          gates_v, slot_v, ids_v, posa_v, posb_v, cnt_v, sem):
        wid = lax.axis_index("s") * 2 + lax.axis_index("c")

        @pl.when(wid < _E)
        def _():
            pltpu.async_copy(gates_hbm, gates_v, sem).wait()
            pltpu.async_copy(slot_hbm, slot_v, sem).wait()
            lane = lax.iota(jnp.int32, 16)
            sent = jnp.zeros((16,), jnp.int32) + _SENT

            def body(i, count):
                ids_v[pl.ds(i * 16, 16)] = jnp.zeros((16,), jnp.int32)
                posa_v[pl.ds(i * 16, 16)] = sent
                posb_v[pl.ds(i * 16, 16)] = sent
                rows = i * 16 + lane
                g16 = plsc.load_gather(gates_v, [rows * _EP + wid])
                s16 = plsc.load_gather(slot_v, [rows * _EP + wid])
                m = g16 > 0.0
                mi = m.astype(jnp.int32)
                pos = plsc.cumsum(mi) + (count - 1)
                first = s16 > 0.0
                flat = pos + wid * _T
                plsc.store_scatter(ids_v, [pos], rows, mask=m)
                plsc.store_scatter(posa_v, [rows], flat, mask=m & first)
                plsc.store_scatter(posb_v, [rows], flat, mask=m & (~first))
                return count + jnp.sum(mi)

            total = lax.fori_loop(0, _T // 16, body, jnp.int32(0))
            cnt_v[...] = jnp.zeros((16,), jnp.int32) + total
            pltpu.sync_copy(ids_v, ids_hbm.at[wid])
            pltpu.sync_copy(posa_v, posa_hbm.at[wid])
            pltpu.sync_copy(posb_v, posb_hbm.at[wid])
            pltpu.sync_copy(cnt_v, cnt_hbm.at[wid])

    return k(gates_flat, slot_flat)


def _gather(x, gates, ids, cnt):
    mesh = plsc.VectorSubcoreMesh(core_axis_name="c", subcore_axis_name="s")
    n_out = [jax.ShapeDtypeStruct((_T, _C), jnp.float32) for _ in range(_E)]
    g_out = [jax.ShapeDtypeStruct((_T, 128), jnp.float32) for _ in range(_E)]

    @functools.partial(
        pl.kernel, mesh=mesh,
        compiler_params=pltpu.CompilerParams(needs_layout_passes=False),
        out_type=n_out + g_out,
        scratch_types=[
            pltpu.VMEM((_CHUNK,), jnp.int32),
            pltpu.VMEM((_CHUNK, _C), jnp.float32),
            pltpu.VMEM((_CHUNK, 128), jnp.float32),
            pltpu.VMEM((16,), jnp.int32),
            pltpu.SemaphoreType.DMA,
        ],
    )
    def k(x_hbm, gates_hbm, ids_hbm, cnt_hbm, *rest):
        outs = rest[:2 * _E]
        idx_v, rows_v, grows_v, cnt_v, sem = rest[2 * _E:]
        wid = lax.axis_index("s") * 2 + lax.axis_index("c")
        for e in range(_E):
            pltpu.sync_copy(cnt_hbm.at[e], cnt_v)
            n = jnp.max(cnt_v[...])

            @pl.when(wid * _CHUNK < n)
            def _():
                pltpu.sync_copy(ids_hbm.at[e, pl.ds(wid * _CHUNK, _CHUNK)],
                                idx_v)
                pltpu.async_copy(x_hbm.at[idx_v], rows_v, sem).wait()
                pltpu.sync_copy(rows_v,
                                outs[e].at[pl.ds(wid * _CHUNK, _CHUNK)])
                pltpu.async_copy(gates_hbm.at[idx_v], grows_v, sem).wait()
                pltpu.sync_copy(grows_v,
                                outs[_E + e].at[pl.ds(wid * _CHUNK, _CHUNK)])

    r = k(x, gates, ids, cnt)
    return r[:_E], r[_E:]


def _combine(posa, posb, yebuf):
    mesh = plsc.VectorSubcoreMesh(core_axis_name="c", subcore_axis_name="s")

    @functools.partial(
        pl.kernel, mesh=mesh,
        compiler_params=pltpu.CompilerParams(needs_layout_passes=False),
        out_type=jax.ShapeDtypeStruct((_T, _C), jnp.float32),
        scratch_types=[
            pltpu.VMEM((_CHUNK,), jnp.int32),
            pltpu.VMEM((_CHUNK,), jnp.int32),
            pltpu.VMEM((_CHUNK,), jnp.int32),
            pltpu.VMEM((_CHUNK, _C), jnp.float32),
            pltpu.VMEM((_CHUNK, _C), jnp.float32),
            pltpu.SemaphoreType.DMA,
            pltpu.SemaphoreType.DMA,
        ],
    )
    def k(posa_hbm, posb_hbm, ye_hbm, out_hbm,
          cand_v, besta_v, bestb_v, rows_a, rows_b, sem, sem2):
        wid = lax.axis_index("s") * 2 + lax.axis_index("c")
        t0 = wid * _CHUNK

        for e in range(_E):
            pltpu.sync_copy(posa_hbm.at[e, pl.ds(t0, _CHUNK)], cand_v)
            for j in range(_CHUNK // 16):
                c16 = cand_v[pl.ds(j * 16, 16)]
                if e == 0:
                    besta_v[pl.ds(j * 16, 16)] = c16
                else:
                    besta_v[pl.ds(j * 16, 16)] = jnp.minimum(
                        besta_v[pl.ds(j * 16, 16)], c16)
        for e in range(_E):
            pltpu.sync_copy(posb_hbm.at[e, pl.ds(t0, _CHUNK)], cand_v)
            for j in range(_CHUNK // 16):
                c16 = cand_v[pl.ds(j * 16, 16)]
                if e == 0:
                    bestb_v[pl.ds(j * 16, 16)] = c16
                else:
                    bestb_v[pl.ds(j * 16, 16)] = jnp.minimum(
                        bestb_v[pl.ds(j * 16, 16)], c16)

        ca = pltpu.async_copy(ye_hbm.at[besta_v], rows_a, sem)
        cb = pltpu.async_copy(ye_hbm.at[bestb_v], rows_b, sem2)
        ca.wait()
        cb.wait()

        def abody(i, carry):
            for j in range(_C // 16):
                rows_a[i, pl.ds(j * 16, 16)] = (
                    rows_a[i, pl.ds(j * 16, 16)]
                    + rows_b[i, pl.ds(j * 16, 16)])
            return carry

        lax.fori_loop(0, _CHUNK, abody, 0)
        pltpu.sync_copy(rows_a, out_hbm.at[pl.ds(t0, _CHUNK)])

    return k(posa, posb, yebuf)


def _mask_rows(y, n, pid, blk):
    ri = jax.lax.broadcasted_iota(jnp.int32, (blk, 1), 0)
    return jnp.where(ri < n - pid * blk, y, 0.0)


def _deep_a_body(cnt_ref, x_ref, w1, b1, w2, b2, lg, lb, out_ref, *, e, blk):
    @pl.when(pl.program_id(0) * blk < cnt_ref[e])
    def _():
        xx = x_ref[0].astype(jnp.bfloat16)
        h = _silu(jnp.dot(xx, w1[...], preferred_element_type=jnp.float32)
                  + b1[...])
        h = jnp.dot(h.astype(jnp.bfloat16), w2[...],
                    preferred_element_type=jnp.float32) + b2[...]
        out_ref[...] = _silu(_lnorm(h, lg[...], lb[...])).astype(jnp.bfloat16)


def _deep_b_body(cnt_ref, x_ref, g_ref, h_ref, w3, b3, ng, nb, out_ref,
                 *, e, blk):
    pid = pl.program_id(0)
    n = cnt_ref[e]

    @pl.when(pid * blk < n)
    def _():
        xx = x_ref[0]
        o = jnp.dot(h_ref[...], w3[...],
                    preferred_element_type=jnp.float32) + b3[...]
        y = _lnorm(xx + o, ng[...], nb[...])
        y = g_ref[0][:, e:e + 1] * y
        out_ref[...] = _mask_rows(y, n, pid, blk)


def _wide_body(cnt_ref, x_ref, g_ref, w1, b1, lg, lb, w2, b2, ng, nb, out_ref,
               *, e, blk):
    pid = pl.program_id(0)
    n = cnt_ref[e]

    @pl.when(pid * blk < n)
    def _():
        xx = x_ref[0]
        h = _gelu(jnp.dot(xx.astype(jnp.bfloat16), w1[...],
                          preferred_element_type=jnp.float32) + b1[...])
        h = _lnorm(h, lg[...], lb[...])
        o = jnp.dot(h.astype(jnp.bfloat16), w2[...],
                    preferred_element_type=jnp.float32) + b2[...]
        y = _lnorm(xx + o, ng[...], nb[...])
        y = g_ref[0][:, e:e + 1] * y
        out_ref[...] = _mask_rows(y, n, pid, blk)


def _hybrid_body(cnt_ref, x_ref, g_ref, p1w1, p1b1, p1w2, p1b2,
                 p2w1, p2b1, p2w2, p2b2, pw1, pw2, pb, ng, nb, out_ref,
                 *, e, blk):
    pid = pl.program_id(0)
    n = cnt_ref[e]

    @pl.when(pid * blk < n)
    def _():
        xx = x_ref[0]
        x16 = xx.astype(jnp.bfloat16)
        h1 = _gelu(jnp.dot(x16, p1w1[...], preferred_element_type=jnp.float32)
                   + p1b1[...])
        o1 = jnp.dot(h1.astype(jnp.bfloat16), p1w2[...],
                     preferred_element_type=jnp.float32) + p1b2[...]
        h2 = _silu(jnp.dot(x16, p2w1[...], preferred_element_type=jnp.float32)
                   + p2b1[...])
        o2 = jnp.dot(h2.astype(jnp.bfloat16), p2w2[...],
                     preferred_element_type=jnp.float32) + p2b2[...]
        o = (jnp.dot(o1.astype(jnp.bfloat16), pw1[...],
                     preferred_element_type=jnp.float32)
             + jnp.dot(o2.astype(jnp.bfloat16), pw2[...],
                       preferred_element_type=jnp.float32) + pb[...])
        y = _lnorm(xx + o, ng[...], nb[...])
        y = g_ref[0][:, e:e + 1] * y
        out_ref[...] = _mask_rows(y, n, pid, blk)


def _ragged_call(body, e, cntv, xg, gg, weights, out_shape, blk,
                 extra=None, with_g=True, out_dtype=jnp.float32,
                 acc=None):
    ins = [xg.reshape(1, *xg.shape)]
    specs = [pl.BlockSpec((1, blk, _C), lambda t, s: (0, t, 0))]
    if with_g:
        ins.append(gg.reshape(1, *gg.shape))
        specs.append(pl.BlockSpec((1, blk, 128), lambda t, s: (0, t, 0)))
    if extra is not None:
        ins.append(extra[0])
        specs.append(pl.BlockSpec((blk, extra[1]), lambda t, s: (t, 0)))
    for w in weights:
        ins.append(w)
        specs.append(pl.BlockSpec(w.shape, lambda t, s, n=w.ndim: (0,) * n))
    nblk = _T // blk
    if acc is not None:
        ins.append(acc)
        specs.append(pl.BlockSpec((blk, out_shape[1]),
                                  lambda t, s, e_=e, nb=nblk: (e_ * nb + t, 0)))
        out_spec = pl.BlockSpec((blk, out_shape[1]),
                                lambda t, s, e_=e, nb=nblk: (e_ * nb + t, 0))
        alias = {len(ins): 0}
    else:
        out_spec = pl.BlockSpec((blk, out_shape[1]), lambda t, s: (t, 0))
        alias = {}
    grid_spec = pltpu.PrefetchScalarGridSpec(
        num_scalar_prefetch=1,
        grid=(nblk,),
        in_specs=specs,
        out_specs=out_spec,
    )
    if acc is not None:
        def wrapped(*refs):
            return body(*refs[:-2], refs[-1], e=e, blk=blk)
    else:
        wrapped = functools.partial(body, e=e, blk=blk)
    return pl.pallas_call(
        wrapped,
        grid_spec=grid_spec,
        out_shape=jax.ShapeDtypeStruct(out_shape, out_dtype),
        input_output_aliases=alias,
    )(cntv, *ins)


def kernel(x, params):
    p = params
    xf = x.reshape(_T, _C)
    et = np.array(_ET)

    tf = p["type_emb2"][jnp.array(et, jnp.int32)]
    tf8 = jnp.concatenate([tf, jnp.zeros((8 - _E, 2 * _C), jnp.float32)], 0)
    w1a = p["route_w1"][:_C]
    w1b = p["route_w1"][_C:]
    rb1 = _row(p["route_b1"])
    rw2p = jnp.zeros((4 * _C, _EP), jnp.float32).at[:, :_E].set(p["route_w2"])
    rb2p = jnp.zeros((1, _EP), jnp.float32).at[0, :_E].set(p["route_b2"])
    temp = jnp.clip(p["temperature"] * (0.95 ** (_T // 100)), 0.05, 3.0)
    norm = jax.random.normal(jax.random.key(42), (_T, _E), jnp.float32)
    norm_p = jnp.concatenate(
        [temp * norm, jnp.zeros((_T, _EP - _E), jnp.float32)], 1)
    bonus = jnp.full((_EP,), _NEG, jnp.float32)
    bonus = bonus.at[:_E].set(0.3 * (et == 1).astype(jnp.float32))
    bonus = _row(bonus)
    nw1p = jnp.zeros((_C, 128), jnp.float32).at[:, :2 * _E].set(p["noise_w1"])
    nb1p = jnp.zeros((1, 128), jnp.float32).at[0, :2 * _E].set(p["noise_b1"])
    nw2p = jnp.zeros((128, _EP), jnp.float32).at[:2 * _E, :_E].set(p["noise_w2"])
    nb2p = jnp.zeros((1, _EP), jnp.float32).at[0, :_E].set(p["noise_b2"])

    ce = pl.pallas_call(
        _prep_body,
        out_shape=jax.ShapeDtypeStruct((8, 4 * _C), jnp.float32),
    )(tf8, w1b, rb1)

    gates, gpad, slot1 = pl.pallas_call(
        _router_body,
        grid=(_T // _BLK,),
        in_specs=[
            pl.BlockSpec((_BLK, _C), lambda t: (t, 0)),
            pl.BlockSpec((_C, 4 * _C), lambda t: (0, 0)),
            pl.BlockSpec((8, 4 * _C), lambda t: (0, 0)),
            pl.BlockSpec((4 * _C, _EP), lambda t: (0, 0)),
            pl.BlockSpec((1, _EP), lambda t: (0, 0)),
            pl.BlockSpec((_C, 128), lambda t: (0, 0)),
            pl.BlockSpec((1, 128), lambda t: (0, 0)),
            pl.BlockSpec((128, _EP), lambda t: (0, 0)),
            pl.BlockSpec((1, _EP), lambda t: (0, 0)),
            pl.BlockSpec((_BLK, _EP), lambda t: (t, 0)),
            pl.BlockSpec((1, _EP), lambda t: (0, 0)),
        ],
        out_specs=[
            pl.BlockSpec((_BLK, _EP), lambda t: (t, 0)),
            pl.BlockSpec((_BLK, 128), lambda t: (t, 0)),
            pl.BlockSpec((_BLK, _EP), lambda t: (t, 0)),
        ],
        out_shape=[
            jax.ShapeDtypeStruct((_T, _EP), jnp.float32),
            jax.ShapeDtypeStruct((_T, 128), jnp.float32),
            jax.ShapeDtypeStruct((_T, _EP), jnp.float32),
        ],
    )(xf, w1a, ce, rw2p, rb2p, nw1p, nb1p, nw2p, nb2p, norm_p, bonus)

    ids, posa, posb, cnt = _listbuild(gates.reshape(-1), slot1.reshape(-1))
    xgs, ggs = _gather(xf, gpad, ids, cnt)
    cntv = cnt[:, 0]

    yebuf = jnp.zeros((_E * _T + 8, _C), jnp.float32)
    ysh = (_E * _T + 8, _C)
    ex = p["experts"]
    for e, t in enumerate(_ET):
        q = ex[e]
        if t == 0:
            hmid = _ragged_call(
                _deep_a_body, e, cntv, xgs[e], None,
                [_b16(q["w1"]), _row(q["b1"]), _b16(q["w2"]), _row(q["b2"]),
                 _row(q["ln_g"]), _row(q["ln_b"])],
                (_T, 4 * _C), 128, with_g=False, out_dtype=jnp.bfloat16)
            yebuf = _ragged_call(
                _deep_b_body, e, cntv, xgs[e], ggs[e],
                [_b16(q["w3"]), _row(q["b3"]), _row(q["ng"]), _row(q["nb"])],
                ysh, _BLK, extra=(hmid, 4 * _C), acc=yebuf)
        elif t == 1:
            yebuf = _ragged_call(
                _wide_body, e, cntv, xgs[e], ggs[e],
                [_b16(q["w1"]), _row(q["b1"]),
                 _row(q["ln_g"]), _row(q["ln_b"]),
                 _b16(q["w2"]), _row(q["b2"]),
                 _row(q["ng"]), _row(q["nb"])],
                ysh, _BLK, acc=yebuf)
        else:
            yebuf = _ragged_call(
                _hybrid_body, e, cntv, xgs[e], ggs[e],
                [_b16(q["p1w1"]), _row(q["p1b1"]),
                 _b16(q["p1w2"]), _row(q["p1b2"]),
                 _b16(q["p2w1"]), _row(q["p2b1"]),
                 _b16(q["p2w2"]), _row(q["p2b2"]),
                 _b16(q["proj_w"][:_C]), _b16(q["proj_w"][_C:]),
                 _row(q["proj_b"]),
                 _row(q["ng"]), _row(q["nb"])],
                ysh, _BLK, acc=yebuf)

    out = _combine(posa, posb, yebuf)

    return out.reshape(1, _T, _C)
